# baseline (speedup 1.0000x reference)
import functools
import math

import jax
import jax.numpy as jnp
from jax import lax
from jax.experimental import pallas as pl
from jax.experimental.pallas import tpu as pltpu
from jax.experimental.pallas import tpu_sc as plsc

VOCAB = 1000000
D_MODEL = 64
BATCH = 4096
SEQ_LEN = 200
CHUNK = 100
CHUNKS_PER_ROW = SEQ_LEN // CHUNK
NC, NS = 2, 16
NW = NC * NS
ROWS_PER_W = BATCH // NW
NBUF = 2
SCALE = math.sqrt(D_MODEL) / SEQ_LEN

BLK_COLS = 256
NBLK_FULL = VOCAB // BLK_COLS
LAST_COLS = VOCAB - NBLK_FULL * BLK_COLS
D_PACK = D_MODEL // 2
BLK_WORDS = BLK_COLS * D_PACK


def _transpose_kernel(t_hbm, out_hbm, stg0, stg1, ost0, ost1, stg_h,
                      isem0, isem1, osem0, osem1):
    w = lax.axis_index("s") * NC + lax.axis_index("c")
    nb = jnp.where(w < NBLK_FULL % NW, NBLK_FULL // NW + 1, NBLK_FULL // NW)
    iota = lax.iota(jnp.int32, 16)
    iota32 = iota * D_PACK
    m_perm = [lax.rem(iota + s, 16) for s in range(16)]
    m_perm32 = [m * D_PACK for m in m_perm]

    def pack(e, o):
        e_i = plsc.bitcast(e, jnp.int32) + 0x8000
        o_i = plsc.bitcast(o, jnp.int32) + 0x8000
        return jnp.bitwise_or(lax.shift_right_logical(e_i, 16),
                              jnp.bitwise_and(o_i, jnp.int32(-65536)))

    def in_slice(j):
        return t_hbm.at[:, pl.ds(j * BLK_COLS, BLK_COLS)]

    def out_slice(j):
        return out_hbm.at[pl.ds(j * BLK_WORDS, BLK_WORDS)]

    def transpose(stg, ost, nchunks):
        def kbody(k, carry):
            for kk in range(nchunks):
                e = stg[2 * k, pl.ds(16 * kk, 16)]
                o = stg[2 * k + 1, pl.ds(16 * kk, 16)]
                idx = iota32 + (kk * 16 * D_PACK) + k
                plsc.store_scatter(ost, [idx], pack(e, o))
            return carry
        lax.fori_loop(0, D_PACK, kbody, 0)

    def transpose_diag(stg, ost):
        @plsc.parallel_loop(0, BLK_COLS // 16)
        def vbody(g):
            v0_32 = g * (16 * D_PACK)
            for k0 in range(0, D_PACK, 16):
                rows_e = (iota + k0) * 2
                rows_o = rows_e + 1
                ve = [plsc.load_gather(stg, [rows_e, m_perm[s] + g * 16])
                      for s in range(16)]
                vo = [plsc.load_gather(stg, [rows_o, m_perm[s] + g * 16])
                      for s in range(16)]
                for s in range(16):
                    oidx = m_perm32[s] + (iota + (v0_32 + k0))
                    plsc.store_scatter(ost, [oidx], pack(ve[s], vo[s]))

    def slot(i, j, stg, ost, isem, osem):
        pltpu.make_async_copy(in_slice(j), stg, isem).wait()

        def _drain_prev_out():
            pltpu.make_async_copy(ost, out_slice(j - 2 * NW), osem).wait()

        def _prefetch_in():
            pltpu.async_copy(in_slice(j + 2 * NW), stg, isem)

        pl.when(i >= 2)(_drain_prev_out)
        transpose_diag(stg, ost)
        pltpu.async_copy(ost, out_slice(j), osem)
        pl.when(i + 2 < nb)(_prefetch_in)

    pltpu.async_copy(in_slice(w), stg0, isem0)
    pltpu.async_copy(in_slice(w + NW), stg1, isem1)

    def body(i, carry):
        j = w + NW * i
        even = lax.rem(i, 2) == 0
        pl.when(even)(lambda: slot(i, j, stg0, ost0, isem0, osem0))
        pl.when(jnp.logical_not(even))(
            lambda: slot(i, j, stg1, ost1, isem1, osem1))
        return carry

    lax.fori_loop(0, nb, body, 0)

    last = nb - 1
    i0 = jnp.where(lax.rem(last, 2) == 0, last, last - 1)
    i1 = jnp.where(lax.rem(last, 2) == 1, last, last - 1)
    pltpu.make_async_copy(ost0, out_slice(w + NW * i0), osem0).wait()
    pltpu.make_async_copy(ost1, out_slice(w + NW * i1), osem1).wait()

    def half_block():
        pltpu.sync_copy(t_hbm.at[:, pl.ds(NBLK_FULL * BLK_COLS, LAST_COLS)],
                        stg_h)
        transpose(stg_h, ost0, LAST_COLS // 16)
        pltpu.sync_copy(ost0.at[pl.ds(0, LAST_COLS * D_PACK)],
                        out_hbm.at[pl.ds(NBLK_FULL * BLK_WORDS,
                                         LAST_COLS * D_PACK)])

    pl.when(w == NBLK_FULL % NW)(half_block)


def _pool_kernel(table_hbm, idx_hbm, out_hbm, idx_v, buf0, buf1, out_v,
                 sem0, sem1):
    wid = lax.axis_index("s") * NC + lax.axis_index("c")
    base = wid * ROWS_PER_W
    bufs = (buf0, buf1)
    sems = (sem0, sem1)
    iota = lax.iota(jnp.int32, 16)

    pltpu.sync_copy(
        idx_hbm.at[pl.ds(base * CHUNKS_PER_ROW, ROWS_PER_W * CHUNKS_PER_ROW)],
        idx_v)

    def fire(b, k):
        r = b * CHUNKS_PER_ROW
        pltpu.async_copy(table_hbm.at[idx_v.at[r]],
                         bufs[k].at[pl.ds(0, CHUNK)], sems[k])
        pltpu.async_copy(table_hbm.at[idx_v.at[r + 1]],
                         bufs[k].at[pl.ds(CHUNK, CHUNK)], sems[k])

    def drain(b, k):
        r = b * CHUNKS_PER_ROW
        pltpu.make_async_copy(table_hbm.at[idx_v.at[r]],
                              bufs[k].at[pl.ds(0, CHUNK)], sems[k]).wait()
        pltpu.make_async_copy(table_hbm.at[idx_v.at[r + 1]],
                              bufs[k].at[pl.ds(CHUNK, CHUNK)], sems[k]).wait()

    def accumulate(b, k):
        buf = bufs[k]

        def body(j, accs):
            new = []
            for h in range(2):
                w = buf[j, pl.ds(16 * h, 16)]
                lo = plsc.bitcast(lax.shift_left(w, 16), jnp.float32)
                hi = plsc.bitcast(jnp.bitwise_and(w, jnp.int32(-65536)),
                                  jnp.float32)
                new.append(accs[2 * h] + lo)
                new.append(accs[2 * h + 1] + hi)
            return tuple(new)

        zeros = tuple(jnp.zeros((16,), jnp.float32) for _ in range(4))
        accs = lax.fori_loop(0, SEQ_LEN, body, zeros)
        brow = jnp.full((16,), b, jnp.int32)
        for h in range(2):
            cols = iota * 2 + 32 * h
            plsc.store_scatter(out_v, [brow, cols], accs[2 * h] * SCALE)
            plsc.store_scatter(out_v, [brow, cols + 1],
                               accs[2 * h + 1] * SCALE)

    for k in range(NBUF):
        fire(k, k)

    def outer(g):
        for k in range(NBUF):
            b = g + k
            drain(b, k)
            accumulate(b, k)
            fire(b + NBUF, k)

    pl.loop(0, ROWS_PER_W - NBUF, step=NBUF)(outer)

    for k in range(NBUF):
        b = ROWS_PER_W - NBUF + k
        drain(b, k)
        accumulate(b, k)

    pltpu.sync_copy(out_v, out_hbm.at[pl.ds(base, ROWS_PER_W)])


@jax.jit
def _pool(src2, embed_weight_t):
    mesh = plsc.VectorSubcoreMesh(core_axis_name="c", subcore_axis_name="s",
                                  num_cores=NC, num_subcores=NS)
    table_flat = pl.kernel(
        _transpose_kernel,
        out_type=jax.ShapeDtypeStruct((VOCAB * D_PACK,), jnp.int32),
        mesh=mesh,
        scratch_types=[
            pltpu.VMEM((D_MODEL, BLK_COLS), jnp.float32),
            pltpu.VMEM((D_MODEL, BLK_COLS), jnp.float32),
            pltpu.VMEM((BLK_WORDS,), jnp.int32),
            pltpu.VMEM((BLK_WORDS,), jnp.int32),
            pltpu.VMEM((D_MODEL, LAST_COLS), jnp.float32),
            pltpu.SemaphoreType.DMA,
            pltpu.SemaphoreType.DMA,
            pltpu.SemaphoreType.DMA,
            pltpu.SemaphoreType.DMA,
        ],
        compiler_params=pltpu.CompilerParams(use_tc_tiling_on_sc=True,
                                             needs_layout_passes=False),
    )(embed_weight_t)
    table_lin = table_flat.reshape(VOCAB, D_PACK)
    return pl.kernel(
        _pool_kernel,
        out_type=jax.ShapeDtypeStruct((BATCH, D_MODEL), jnp.float32),
        mesh=mesh,
        scratch_types=[
            pltpu.VMEM((ROWS_PER_W * CHUNKS_PER_ROW, CHUNK), jnp.int32),
            pltpu.VMEM((SEQ_LEN, D_PACK), jnp.int32),
            pltpu.VMEM((SEQ_LEN, D_PACK), jnp.int32),
            pltpu.VMEM((ROWS_PER_W, D_MODEL), jnp.float32),
            pltpu.SemaphoreType.DMA,
            pltpu.SemaphoreType.DMA,
        ],
        compiler_params=pltpu.CompilerParams(use_tc_tiling_on_sc=False,
                                             needs_layout_passes=False),
    )(table_lin, src2)


def kernel(src, embed_weight):
    src2 = src.astype(jnp.int32).reshape(BATCH * CHUNKS_PER_ROW, CHUNK)
    return _pool(src2, embed_weight.T)


# speedup vs baseline: 1.3815x; 1.3815x over previous
import functools
import math

import jax
import jax.numpy as jnp
from jax import lax
from jax.experimental import pallas as pl
from jax.experimental.pallas import tpu as pltpu
from jax.experimental.pallas import tpu_sc as plsc

VOCAB = 1000000
D_MODEL = 64
BATCH = 4096
SEQ_LEN = 200
CHUNK = 100
CHUNKS_PER_ROW = SEQ_LEN // CHUNK
NC, NS = 2, 16
NW = NC * NS
ROWS_PER_W = BATCH // NW
NBUF = 2
SCALE = math.sqrt(D_MODEL) / SEQ_LEN

BLK_COLS = 256
NBLK_FULL = VOCAB // BLK_COLS
LAST_COLS = VOCAB - NBLK_FULL * BLK_COLS
D_PACK = D_MODEL // 2
BLK_WORDS = BLK_COLS * D_PACK


def _transpose_kernel(t_hbm, out_hbm, stg0, stg1, ost0, ost1, stg_h,
                      isem0, isem1, osem0, osem1):
    w = lax.axis_index("s") * NC + lax.axis_index("c")
    nb = jnp.where(w < NBLK_FULL % NW, NBLK_FULL // NW + 1, NBLK_FULL // NW)
    iota = lax.iota(jnp.int32, 16)
    iota32 = iota * D_PACK
    m_perm = [lax.rem(iota + s, 16) for s in range(16)]
    m_perm32 = [m * D_PACK for m in m_perm]

    def pack(e, o):
        e_i = plsc.bitcast(e, jnp.int32) + 0x8000
        o_i = plsc.bitcast(o, jnp.int32) + 0x8000
        return jnp.bitwise_or(lax.shift_right_logical(e_i, 16),
                              jnp.bitwise_and(o_i, jnp.int32(-65536)))

    def in_slice(j):
        return t_hbm.at[:, pl.ds(j * BLK_COLS, BLK_COLS)]

    def out_slice(j):
        return out_hbm.at[pl.ds(j * BLK_WORDS, BLK_WORDS)]

    def transpose(stg, ost, nchunks):
        def kbody(k, carry):
            for kk in range(nchunks):
                e = stg[2 * k, pl.ds(16 * kk, 16)]
                o = stg[2 * k + 1, pl.ds(16 * kk, 16)]
                idx = iota32 + (kk * 16 * D_PACK) + k
                plsc.store_scatter(ost, [idx], pack(e, o))
            return carry
        lax.fori_loop(0, D_PACK, kbody, 0)

    def transpose_diag(stg, ost):
        @plsc.parallel_loop(0, BLK_COLS // 16)
        def vbody(g):
            v0_32 = g * (16 * D_PACK)
            for k0 in range(0, D_PACK, 16):
                rows_e = (iota + k0) * 2
                rows_o = rows_e + 1
                for sb in range(0, 16, 4):
                    ve = [plsc.load_gather(stg, [rows_e, m_perm[s] + g * 16])
                          for s in range(sb, sb + 4)]
                    vo = [plsc.load_gather(stg, [rows_o, m_perm[s] + g * 16])
                          for s in range(sb, sb + 4)]
                    for t, s in enumerate(range(sb, sb + 4)):
                        oidx = m_perm32[s] + (iota + (v0_32 + k0))
                        plsc.store_scatter(ost, [oidx], pack(ve[t], vo[t]))

    def slot(i, j, stg, ost, isem, osem):
        pltpu.make_async_copy(in_slice(j), stg, isem).wait()

        def _drain_prev_out():
            pltpu.make_async_copy(ost, out_slice(j - 2 * NW), osem).wait()

        def _prefetch_in():
            pltpu.async_copy(in_slice(j + 2 * NW), stg, isem)

        pl.when(i >= 2)(_drain_prev_out)
        transpose_diag(stg, ost)
        pltpu.async_copy(ost, out_slice(j), osem)
        pl.when(i + 2 < nb)(_prefetch_in)

    pltpu.async_copy(in_slice(w), stg0, isem0)
    pltpu.async_copy(in_slice(w + NW), stg1, isem1)

    def body(i, carry):
        j = w + NW * i
        even = lax.rem(i, 2) == 0
        pl.when(even)(lambda: slot(i, j, stg0, ost0, isem0, osem0))
        pl.when(jnp.logical_not(even))(
            lambda: slot(i, j, stg1, ost1, isem1, osem1))
        return carry

    lax.fori_loop(0, nb, body, 0)

    last = nb - 1
    i0 = jnp.where(lax.rem(last, 2) == 0, last, last - 1)
    i1 = jnp.where(lax.rem(last, 2) == 1, last, last - 1)
    pltpu.make_async_copy(ost0, out_slice(w + NW * i0), osem0).wait()
    pltpu.make_async_copy(ost1, out_slice(w + NW * i1), osem1).wait()

    def half_block():
        pltpu.sync_copy(t_hbm.at[:, pl.ds(NBLK_FULL * BLK_COLS, LAST_COLS)],
                        stg_h)
        transpose(stg_h, ost0, LAST_COLS // 16)
        pltpu.sync_copy(ost0.at[pl.ds(0, LAST_COLS * D_PACK)],
                        out_hbm.at[pl.ds(NBLK_FULL * BLK_WORDS,
                                         LAST_COLS * D_PACK)])

    pl.when(w == NBLK_FULL % NW)(half_block)


def _pool_kernel(table_hbm, idx_hbm, out_hbm, idx_v, buf0, buf1, out_v,
                 sem0, sem1):
    wid = lax.axis_index("s") * NC + lax.axis_index("c")
    base = wid * ROWS_PER_W
    bufs = (buf0, buf1)
    sems = (sem0, sem1)
    iota = lax.iota(jnp.int32, 16)

    pltpu.sync_copy(
        idx_hbm.at[pl.ds(base * CHUNKS_PER_ROW, ROWS_PER_W * CHUNKS_PER_ROW)],
        idx_v)

    def fire(b, k):
        r = b * CHUNKS_PER_ROW
        pltpu.async_copy(table_hbm.at[idx_v.at[r]],
                         bufs[k].at[pl.ds(0, CHUNK)], sems[k])
        pltpu.async_copy(table_hbm.at[idx_v.at[r + 1]],
                         bufs[k].at[pl.ds(CHUNK, CHUNK)], sems[k])

    def drain(b, k):
        r = b * CHUNKS_PER_ROW
        pltpu.make_async_copy(table_hbm.at[idx_v.at[r]],
                              bufs[k].at[pl.ds(0, CHUNK)], sems[k]).wait()
        pltpu.make_async_copy(table_hbm.at[idx_v.at[r + 1]],
                              bufs[k].at[pl.ds(CHUNK, CHUNK)], sems[k]).wait()

    def accumulate(b, k):
        buf = bufs[k]

        def body(j, accs):
            new = []
            for h in range(2):
                w = buf[j, pl.ds(16 * h, 16)]
                lo = plsc.bitcast(lax.shift_left(w, 16), jnp.float32)
                hi = plsc.bitcast(jnp.bitwise_and(w, jnp.int32(-65536)),
                                  jnp.float32)
                new.append(accs[2 * h] + lo)
                new.append(accs[2 * h + 1] + hi)
            return tuple(new)

        zeros = tuple(jnp.zeros((16,), jnp.float32) for _ in range(4))
        accs = lax.fori_loop(0, SEQ_LEN, body, zeros)
        brow = jnp.full((16,), b, jnp.int32)
        for h in range(2):
            cols = iota * 2 + 32 * h
            plsc.store_scatter(out_v, [brow, cols], accs[2 * h] * SCALE)
            plsc.store_scatter(out_v, [brow, cols + 1],
                               accs[2 * h + 1] * SCALE)

    for k in range(NBUF):
        fire(k, k)

    def outer(g):
        for k in range(NBUF):
            b = g + k
            drain(b, k)
            accumulate(b, k)
            fire(b + NBUF, k)

    pl.loop(0, ROWS_PER_W - NBUF, step=NBUF)(outer)

    for k in range(NBUF):
        b = ROWS_PER_W - NBUF + k
        drain(b, k)
        accumulate(b, k)

    pltpu.sync_copy(out_v, out_hbm.at[pl.ds(base, ROWS_PER_W)])


@jax.jit
def _pool(src2, embed_weight_t):
    mesh = plsc.VectorSubcoreMesh(core_axis_name="c", subcore_axis_name="s",
                                  num_cores=NC, num_subcores=NS)
    table_flat = pl.kernel(
        _transpose_kernel,
        out_type=jax.ShapeDtypeStruct((VOCAB * D_PACK,), jnp.int32),
        mesh=mesh,
        scratch_types=[
            pltpu.VMEM((D_MODEL, BLK_COLS), jnp.float32),
            pltpu.VMEM((D_MODEL, BLK_COLS), jnp.float32),
            pltpu.VMEM((BLK_WORDS,), jnp.int32),
            pltpu.VMEM((BLK_WORDS,), jnp.int32),
            pltpu.VMEM((D_MODEL, LAST_COLS), jnp.float32),
            pltpu.SemaphoreType.DMA,
            pltpu.SemaphoreType.DMA,
            pltpu.SemaphoreType.DMA,
            pltpu.SemaphoreType.DMA,
        ],
        compiler_params=pltpu.CompilerParams(use_tc_tiling_on_sc=True,
                                             needs_layout_passes=False),
    )(embed_weight_t)
    table_lin = table_flat.reshape(VOCAB, D_PACK)
    return pl.kernel(
        _pool_kernel,
        out_type=jax.ShapeDtypeStruct((BATCH, D_MODEL), jnp.float32),
        mesh=mesh,
        scratch_types=[
            pltpu.VMEM((ROWS_PER_W * CHUNKS_PER_ROW, CHUNK), jnp.int32),
            pltpu.VMEM((SEQ_LEN, D_PACK), jnp.int32),
            pltpu.VMEM((SEQ_LEN, D_PACK), jnp.int32),
            pltpu.VMEM((ROWS_PER_W, D_MODEL), jnp.float32),
            pltpu.SemaphoreType.DMA,
            pltpu.SemaphoreType.DMA,
        ],
        compiler_params=pltpu.CompilerParams(use_tc_tiling_on_sc=False,
                                             needs_layout_passes=False),
    )(table_lin, src2)


def kernel(src, embed_weight):
    src2 = src.astype(jnp.int32).reshape(BATCH * CHUNKS_PER_ROW, CHUNK)
    return _pool(src2, embed_weight.T)


# speedup vs baseline: 1.4517x; 1.0508x over previous
import functools
import math

import jax
import jax.numpy as jnp
from jax import lax
from jax.experimental import pallas as pl
from jax.experimental.pallas import tpu as pltpu
from jax.experimental.pallas import tpu_sc as plsc

VOCAB = 1000000
D_MODEL = 64
BATCH = 4096
SEQ_LEN = 200
CHUNK = 100
CHUNKS_PER_ROW = SEQ_LEN // CHUNK
NC, NS = 2, 16
NW = NC * NS
ROWS_PER_W = BATCH // NW
NBUF = 2
SCALE = math.sqrt(D_MODEL) / SEQ_LEN

BLK_COLS = 384
NBLK_FULL = VOCAB // BLK_COLS
LAST_COLS = VOCAB - NBLK_FULL * BLK_COLS
D_PACK = D_MODEL // 2
BLK_WORDS = BLK_COLS * D_PACK


def _transpose_kernel(t_hbm, out_hbm, stg0, stg1, ost0, ost1, stg_h,
                      isem0, isem1, osem0, osem1):
    w = lax.axis_index("s") * NC + lax.axis_index("c")
    nb = jnp.where(w < NBLK_FULL % NW, NBLK_FULL // NW + 1, NBLK_FULL // NW)
    iota = lax.iota(jnp.int32, 16)
    iota32 = iota * D_PACK
    m_perm = [lax.rem(iota + s, 16) for s in range(16)]
    m_perm32 = [m * D_PACK for m in m_perm]

    def pack(e, o):
        e_i = plsc.bitcast(e, jnp.int32) + 0x8000
        o_i = plsc.bitcast(o, jnp.int32) + 0x8000
        return jnp.bitwise_or(lax.shift_right_logical(e_i, 16),
                              jnp.bitwise_and(o_i, jnp.int32(-65536)))

    def in_slice(j):
        return t_hbm.at[:, pl.ds(j * BLK_COLS, BLK_COLS)]

    def out_slice(j):
        return out_hbm.at[pl.ds(j * BLK_WORDS, BLK_WORDS)]

    def transpose(stg, ost, nchunks):
        def kbody(k, carry):
            for kk in range(nchunks):
                e = stg[2 * k, pl.ds(16 * kk, 16)]
                o = stg[2 * k + 1, pl.ds(16 * kk, 16)]
                idx = iota32 + (kk * 16 * D_PACK) + k
                plsc.store_scatter(ost, [idx], pack(e, o))
            return carry
        lax.fori_loop(0, D_PACK, kbody, 0)

    def transpose_diag(stg, ost):
        @plsc.parallel_loop(0, BLK_COLS // 16)
        def vbody(g):
            v0_32 = g * (16 * D_PACK)
            for k0 in range(0, D_PACK, 16):
                rows_e = (iota + k0) * 2
                rows_o = rows_e + 1
                for sb in range(0, 16, 4):
                    ve = [plsc.load_gather(stg, [rows_e, m_perm[s] + g * 16])
                          for s in range(sb, sb + 4)]
                    vo = [plsc.load_gather(stg, [rows_o, m_perm[s] + g * 16])
                          for s in range(sb, sb + 4)]
                    for t, s in enumerate(range(sb, sb + 4)):
                        oidx = m_perm32[s] + (iota + (v0_32 + k0))
                        plsc.store_scatter(ost, [oidx], pack(ve[t], vo[t]))

    def slot(i, j, stg, ost, isem, osem):
        pltpu.make_async_copy(in_slice(j), stg, isem).wait()

        def _drain_prev_out():
            pltpu.make_async_copy(ost, out_slice(j - 2 * NW), osem).wait()

        def _prefetch_in():
            pltpu.async_copy(in_slice(j + 2 * NW), stg, isem)

        pl.when(i >= 2)(_drain_prev_out)
        transpose_diag(stg, ost)
        pltpu.async_copy(ost, out_slice(j), osem)
        pl.when(i + 2 < nb)(_prefetch_in)

    pltpu.async_copy(in_slice(w), stg0, isem0)
    pltpu.async_copy(in_slice(w + NW), stg1, isem1)

    def body(i, carry):
        j = w + NW * i
        even = lax.rem(i, 2) == 0
        pl.when(even)(lambda: slot(i, j, stg0, ost0, isem0, osem0))
        pl.when(jnp.logical_not(even))(
            lambda: slot(i, j, stg1, ost1, isem1, osem1))
        return carry

    lax.fori_loop(0, nb, body, 0)

    last = nb - 1
    i0 = jnp.where(lax.rem(last, 2) == 0, last, last - 1)
    i1 = jnp.where(lax.rem(last, 2) == 1, last, last - 1)
    pltpu.make_async_copy(ost0, out_slice(w + NW * i0), osem0).wait()
    pltpu.make_async_copy(ost1, out_slice(w + NW * i1), osem1).wait()

    def half_block():
        pltpu.sync_copy(t_hbm.at[:, pl.ds(NBLK_FULL * BLK_COLS, LAST_COLS)],
                        stg_h)
        transpose(stg_h, ost0, LAST_COLS // 16)
        pltpu.sync_copy(ost0.at[pl.ds(0, LAST_COLS * D_PACK)],
                        out_hbm.at[pl.ds(NBLK_FULL * BLK_WORDS,
                                         LAST_COLS * D_PACK)])

    pl.when(w == NBLK_FULL % NW)(half_block)


def _pool_kernel(table_hbm, idx_hbm, out_hbm, idx_v, buf0, buf1, out_v,
                 sem0, sem1):
    wid = lax.axis_index("s") * NC + lax.axis_index("c")
    base = wid * ROWS_PER_W
    bufs = (buf0, buf1)
    sems = (sem0, sem1)
    iota = lax.iota(jnp.int32, 16)

    pltpu.sync_copy(
        idx_hbm.at[pl.ds(base * CHUNKS_PER_ROW, ROWS_PER_W * CHUNKS_PER_ROW)],
        idx_v)

    def fire(b, k):
        r = b * CHUNKS_PER_ROW
        pltpu.async_copy(table_hbm.at[idx_v.at[r]],
                         bufs[k].at[pl.ds(0, CHUNK)], sems[k])
        pltpu.async_copy(table_hbm.at[idx_v.at[r + 1]],
                         bufs[k].at[pl.ds(CHUNK, CHUNK)], sems[k])

    def drain(b, k):
        r = b * CHUNKS_PER_ROW
        pltpu.make_async_copy(table_hbm.at[idx_v.at[r]],
                              bufs[k].at[pl.ds(0, CHUNK)], sems[k]).wait()
        pltpu.make_async_copy(table_hbm.at[idx_v.at[r + 1]],
                              bufs[k].at[pl.ds(CHUNK, CHUNK)], sems[k]).wait()

    def accumulate(b, k):
        buf = bufs[k]

        def body(j, accs):
            new = []
            for h in range(2):
                w = buf[j, pl.ds(16 * h, 16)]
                lo = plsc.bitcast(lax.shift_left(w, 16), jnp.float32)
                hi = plsc.bitcast(jnp.bitwise_and(w, jnp.int32(-65536)),
                                  jnp.float32)
                new.append(accs[2 * h] + lo)
                new.append(accs[2 * h + 1] + hi)
            return tuple(new)

        zeros = tuple(jnp.zeros((16,), jnp.float32) for _ in range(4))
        accs = lax.fori_loop(0, SEQ_LEN, body, zeros)
        brow = jnp.full((16,), b, jnp.int32)
        for h in range(2):
            cols = iota * 2 + 32 * h
            plsc.store_scatter(out_v, [brow, cols], accs[2 * h] * SCALE)
            plsc.store_scatter(out_v, [brow, cols + 1],
                               accs[2 * h + 1] * SCALE)

    for k in range(NBUF):
        fire(k, k)

    def outer(g):
        for k in range(NBUF):
            b = g + k
            drain(b, k)
            accumulate(b, k)
            fire(b + NBUF, k)

    pl.loop(0, ROWS_PER_W - NBUF, step=NBUF)(outer)

    for k in range(NBUF):
        b = ROWS_PER_W - NBUF + k
        drain(b, k)
        accumulate(b, k)

    pltpu.sync_copy(out_v, out_hbm.at[pl.ds(base, ROWS_PER_W)])


@jax.jit
def _pool(src2, embed_weight_t):
    mesh = plsc.VectorSubcoreMesh(core_axis_name="c", subcore_axis_name="s",
                                  num_cores=NC, num_subcores=NS)
    table_flat = pl.kernel(
        _transpose_kernel,
        out_type=jax.ShapeDtypeStruct((VOCAB * D_PACK,), jnp.int32),
        mesh=mesh,
        scratch_types=[
            pltpu.VMEM((D_MODEL, BLK_COLS), jnp.float32),
            pltpu.VMEM((D_MODEL, BLK_COLS), jnp.float32),
            pltpu.VMEM((BLK_WORDS,), jnp.int32),
            pltpu.VMEM((BLK_WORDS,), jnp.int32),
            pltpu.VMEM((D_MODEL, LAST_COLS), jnp.float32),
            pltpu.SemaphoreType.DMA,
            pltpu.SemaphoreType.DMA,
            pltpu.SemaphoreType.DMA,
            pltpu.SemaphoreType.DMA,
        ],
        compiler_params=pltpu.CompilerParams(use_tc_tiling_on_sc=True,
                                             needs_layout_passes=False),
    )(embed_weight_t)
    table_lin = table_flat.reshape(VOCAB, D_PACK)
    return pl.kernel(
        _pool_kernel,
        out_type=jax.ShapeDtypeStruct((BATCH, D_MODEL), jnp.float32),
        mesh=mesh,
        scratch_types=[
            pltpu.VMEM((ROWS_PER_W * CHUNKS_PER_ROW, CHUNK), jnp.int32),
            pltpu.VMEM((SEQ_LEN, D_PACK), jnp.int32),
            pltpu.VMEM((SEQ_LEN, D_PACK), jnp.int32),
            pltpu.VMEM((ROWS_PER_W, D_MODEL), jnp.float32),
            pltpu.SemaphoreType.DMA,
            pltpu.SemaphoreType.DMA,
        ],
        compiler_params=pltpu.CompilerParams(use_tc_tiling_on_sc=False,
                                             needs_layout_passes=False),
    )(table_lin, src2)


def kernel(src, embed_weight):
    src2 = src.astype(jnp.int32).reshape(BATCH * CHUNKS_PER_ROW, CHUNK)
    return _pool(src2, embed_weight.T)


# speedup vs baseline: 1.5938x; 1.0979x over previous
import functools
import math

import jax
import jax.numpy as jnp
from jax import lax
from jax.experimental import pallas as pl
from jax.experimental.pallas import tpu as pltpu
from jax.experimental.pallas import tpu_sc as plsc

VOCAB = 1000000
D_MODEL = 64
BATCH = 4096
SEQ_LEN = 200
CHUNK = 100
CHUNKS_PER_ROW = SEQ_LEN // CHUNK
NC, NS = 2, 16
NW = NC * NS
ROWS_PER_W = BATCH // NW
NBUF = 2
SCALE = math.sqrt(D_MODEL) / SEQ_LEN

BLK_COLS = 384
NBLK_FULL = VOCAB // BLK_COLS
LAST_COLS = VOCAB - NBLK_FULL * BLK_COLS
D_PACK = D_MODEL // 2
BLK_WORDS = BLK_COLS * D_PACK


def _transpose_kernel(t_hbm, out_hbm, stg0, stg1, ost0, ost1, stg_h,
                      isem0, isem1, osem0, osem1):
    w = lax.axis_index("s") * NC + lax.axis_index("c")
    nb = jnp.where(w < NBLK_FULL % NW, NBLK_FULL // NW + 1, NBLK_FULL // NW)
    iota = lax.iota(jnp.int32, 16)
    iota32 = iota * D_PACK
    m_perm = [lax.rem(iota + s, 16) for s in range(16)]
    m_perm32 = [m * D_PACK for m in m_perm]

    def pack(e, o):
        e_i = plsc.bitcast(e, jnp.int32) + 0x8000
        o_i = plsc.bitcast(o, jnp.int32) + 0x8000
        return jnp.bitwise_or(lax.shift_right_logical(e_i, 16),
                              jnp.bitwise_and(o_i, jnp.int32(-65536)))

    def in_slice(j):
        return t_hbm.at[:, pl.ds(j * BLK_COLS, BLK_COLS)]

    def out_slice(j):
        return out_hbm.at[pl.ds(j * BLK_WORDS, BLK_WORDS)]

    def transpose(stg, ost, nchunks):
        def kbody(k, carry):
            for kk in range(nchunks):
                e = stg[2 * k, pl.ds(16 * kk, 16)]
                o = stg[2 * k + 1, pl.ds(16 * kk, 16)]
                idx = iota32 + (kk * 16 * D_PACK) + k
                plsc.store_scatter(ost, [idx], pack(e, o))
            return carry
        lax.fori_loop(0, D_PACK, kbody, 0)

    def transpose_diag(stg, ost):
        @plsc.parallel_loop(0, BLK_COLS // 16)
        def vbody(g):
            v0_32 = g * (16 * D_PACK)
            for k0 in range(0, D_PACK, 16):
                rows_e = (iota + k0) * 2
                rows_o = rows_e + 1
                for sb in range(0, 16, 4):
                    ve = [plsc.load_gather(stg, [rows_e, m_perm[s] + g * 16])
                          for s in range(sb, sb + 4)]
                    vo = [plsc.load_gather(stg, [rows_o, m_perm[s] + g * 16])
                          for s in range(sb, sb + 4)]
                    for t, s in enumerate(range(sb, sb + 4)):
                        oidx = m_perm32[s] + (iota + (v0_32 + k0))
                        plsc.store_scatter(ost, [oidx], pack(ve[t], vo[t]))

    def slot(i, j, stg, ost, isem, osem):
        pltpu.make_async_copy(in_slice(j), stg, isem).wait()

        def _drain_prev_out():
            pltpu.make_async_copy(ost, out_slice(j - 2 * NW), osem).wait()

        def _prefetch_in():
            pltpu.async_copy(in_slice(j + 2 * NW), stg, isem)

        pl.when(i >= 2)(_drain_prev_out)
        transpose_diag(stg, ost)
        pltpu.async_copy(ost, out_slice(j), osem)
        pl.when(i + 2 < nb)(_prefetch_in)

    pltpu.async_copy(in_slice(w), stg0, isem0)
    pltpu.async_copy(in_slice(w + NW), stg1, isem1)

    def body(i, carry):
        j = w + NW * i
        even = lax.rem(i, 2) == 0
        pl.when(even)(lambda: slot(i, j, stg0, ost0, isem0, osem0))
        pl.when(jnp.logical_not(even))(
            lambda: slot(i, j, stg1, ost1, isem1, osem1))
        return carry

    lax.fori_loop(0, nb, body, 0)

    last = nb - 1
    i0 = jnp.where(lax.rem(last, 2) == 0, last, last - 1)
    i1 = jnp.where(lax.rem(last, 2) == 1, last, last - 1)
    pltpu.make_async_copy(ost0, out_slice(w + NW * i0), osem0).wait()
    pltpu.make_async_copy(ost1, out_slice(w + NW * i1), osem1).wait()

    def half_block():
        pltpu.sync_copy(t_hbm.at[:, pl.ds(NBLK_FULL * BLK_COLS, LAST_COLS)],
                        stg_h)
        transpose(stg_h, ost0, LAST_COLS // 16)
        pltpu.sync_copy(ost0.at[pl.ds(0, LAST_COLS * D_PACK)],
                        out_hbm.at[pl.ds(NBLK_FULL * BLK_WORDS,
                                         LAST_COLS * D_PACK)])

    pl.when(w == NBLK_FULL % NW)(half_block)


def _pool_kernel(table_hbm, idx_hbm, out_hbm, idx_v, buf0, buf1, out_v,
                 sem0, sem1):
    wid = lax.axis_index("s") * NC + lax.axis_index("c")
    base = wid * ROWS_PER_W
    bufs = (buf0, buf1)
    sems = (sem0, sem1)
    iota = lax.iota(jnp.int32, 16)

    pltpu.sync_copy(
        idx_hbm.at[pl.ds(base * CHUNKS_PER_ROW, ROWS_PER_W * CHUNKS_PER_ROW)],
        idx_v)

    def fire(b, k):
        r = b * CHUNKS_PER_ROW
        pltpu.async_copy(table_hbm.at[idx_v.at[r]],
                         bufs[k].at[pl.ds(0, CHUNK)], sems[k])
        pltpu.async_copy(table_hbm.at[idx_v.at[r + 1]],
                         bufs[k].at[pl.ds(CHUNK, CHUNK)], sems[k])

    def drain(b, k):
        r = b * CHUNKS_PER_ROW
        pltpu.make_async_copy(table_hbm.at[idx_v.at[r]],
                              bufs[k].at[pl.ds(0, CHUNK)], sems[k]).wait()
        pltpu.make_async_copy(table_hbm.at[idx_v.at[r + 1]],
                              bufs[k].at[pl.ds(CHUNK, CHUNK)], sems[k]).wait()

    def accumulate(b, k):
        buf = bufs[k]

        def body(jj, accs):
            new = list(accs)
            for r in range(2):
                j = jj * 2 + r
                for h in range(2):
                    w = buf[j, pl.ds(16 * h, 16)]
                    lo = plsc.bitcast(lax.shift_left(w, 16), jnp.float32)
                    hi = plsc.bitcast(w, jnp.float32)
                    i = 4 * r + 2 * h
                    new[i] = new[i] + lo
                    new[i + 1] = new[i + 1] + hi
            return tuple(new)

        zeros = tuple(jnp.zeros((16,), jnp.float32) for _ in range(8))
        accs = lax.fori_loop(0, SEQ_LEN // 2, body, zeros)
        brow = jnp.full((16,), b, jnp.int32)
        for h in range(2):
            cols = iota * 2 + 32 * h
            plsc.store_scatter(out_v, [brow, cols],
                               (accs[2 * h] + accs[4 + 2 * h]) * SCALE)
            plsc.store_scatter(out_v, [brow, cols + 1],
                               (accs[2 * h + 1] + accs[5 + 2 * h]) * SCALE)

    for k in range(NBUF):
        fire(k, k)

    def outer(g):
        for k in range(NBUF):
            b = g + k
            drain(b, k)
            accumulate(b, k)
            fire(b + NBUF, k)

    pl.loop(0, ROWS_PER_W - NBUF, step=NBUF)(outer)

    for k in range(NBUF):
        b = ROWS_PER_W - NBUF + k
        drain(b, k)
        accumulate(b, k)

    pltpu.sync_copy(out_v, out_hbm.at[pl.ds(base, ROWS_PER_W)])


@jax.jit
def _pool(src2, embed_weight_t):
    mesh = plsc.VectorSubcoreMesh(core_axis_name="c", subcore_axis_name="s",
                                  num_cores=NC, num_subcores=NS)
    table_flat = pl.kernel(
        _transpose_kernel,
        out_type=jax.ShapeDtypeStruct((VOCAB * D_PACK,), jnp.int32),
        mesh=mesh,
        scratch_types=[
            pltpu.VMEM((D_MODEL, BLK_COLS), jnp.float32),
            pltpu.VMEM((D_MODEL, BLK_COLS), jnp.float32),
            pltpu.VMEM((BLK_WORDS,), jnp.int32),
            pltpu.VMEM((BLK_WORDS,), jnp.int32),
            pltpu.VMEM((D_MODEL, LAST_COLS), jnp.float32),
            pltpu.SemaphoreType.DMA,
            pltpu.SemaphoreType.DMA,
            pltpu.SemaphoreType.DMA,
            pltpu.SemaphoreType.DMA,
        ],
        compiler_params=pltpu.CompilerParams(use_tc_tiling_on_sc=True,
                                             needs_layout_passes=False),
    )(embed_weight_t)
    table_lin = table_flat.reshape(VOCAB, D_PACK)
    return pl.kernel(
        _pool_kernel,
        out_type=jax.ShapeDtypeStruct((BATCH, D_MODEL), jnp.float32),
        mesh=mesh,
        scratch_types=[
            pltpu.VMEM((ROWS_PER_W * CHUNKS_PER_ROW, CHUNK), jnp.int32),
            pltpu.VMEM((SEQ_LEN, D_PACK), jnp.int32),
            pltpu.VMEM((SEQ_LEN, D_PACK), jnp.int32),
            pltpu.VMEM((ROWS_PER_W, D_MODEL), jnp.float32),
            pltpu.SemaphoreType.DMA,
            pltpu.SemaphoreType.DMA,
        ],
        compiler_params=pltpu.CompilerParams(use_tc_tiling_on_sc=False,
                                             needs_layout_passes=False),
    )(table_lin, src2)


def kernel(src, embed_weight):
    src2 = src.astype(jnp.int32).reshape(BATCH * CHUNKS_PER_ROW, CHUNK)
    return _pool(src2, embed_weight.T)
